# Initial kernel scaffold; baseline (speedup 1.0000x reference)
import jax
import jax.numpy as jnp
from jax.experimental import pallas as pl


def kernel(x, edge_index, batch, W1_0, b1_0, W2_0, b2_0, W1_1, b1_1, W2_1, b2_1, W1_2, b1_2, W2_2, b2_2):
    raise NotImplementedError("write your pallas kernel here")


# speedup vs baseline: 5.0052x; 5.0052x over previous
import functools

import jax
import jax.numpy as jnp
from jax import lax
from jax.experimental import pallas as pl
from jax.experimental.pallas import tpu as pltpu
from jax.experimental.pallas import tpu_sc as plsc

N = 10000
E = 320000
D = 128

NC = 2
NS = 16
NW = NC * NS
EDGES_PER_W = E // NW
CHUNK = 80
NCHUNK = EDGES_PER_W // CHUNK
ROWS_PER_TILE = 624
ROWS_TAIL = N - NS * ROWS_PER_TILE

_mesh = plsc.VectorSubcoreMesh(core_axis_name="c", subcore_axis_name="s")


@functools.partial(
    pl.kernel,
    out_type=jax.ShapeDtypeStruct((NC * N, D), jnp.float32),
    mesh=_mesh,
    scratch_types=[
        pltpu.VMEM((CHUNK,), jnp.int32),
        pltpu.VMEM((CHUNK,), jnp.int32),
        pltpu.VMEM((CHUNK, D), jnp.float32),
        pltpu.VMEM_SHARED((N, D), jnp.float32),
        pltpu.SemaphoreType.DMA,
    ],
)
def _sc_agg(h_hbm, src_hbm, dst_hbm, zeros_hbm, out_hbm,
            sidx, didx, rows, acc, gsem):
    c = lax.axis_index("c")
    s = lax.axis_index("s")
    wid = c * NS + s
    ebase = wid * EDGES_PER_W

    r0 = s * ROWS_PER_TILE
    pltpu.sync_copy(zeros_hbm.at[pl.ds(r0, ROWS_PER_TILE)],
                    acc.at[pl.ds(r0, ROWS_PER_TILE)])

    @pl.when(s == 0)
    def _zero_tail():
        pltpu.sync_copy(zeros_hbm.at[pl.ds(NS * ROWS_PER_TILE, ROWS_TAIL)],
                        acc.at[pl.ds(NS * ROWS_PER_TILE, ROWS_TAIL)])

    plsc.subcore_barrier()

    def step(i, carry):
        off = pl.multiple_of(ebase + i * CHUNK, 8)
        pltpu.sync_copy(src_hbm.at[pl.ds(off, CHUNK)], sidx)
        pltpu.sync_copy(dst_hbm.at[pl.ds(off, CHUNK)], didx)
        pltpu.async_copy(h_hbm.at[sidx], rows, gsem).wait()
        pltpu.sync_copy(rows, acc.at[didx], add=True)
        return carry

    lax.fori_loop(0, NCHUNK, step, 0)
    plsc.subcore_barrier()

    obase = c * N + r0
    pltpu.sync_copy(acc.at[pl.ds(r0, ROWS_PER_TILE)],
                    out_hbm.at[pl.ds(obase, ROWS_PER_TILE)])

    @pl.when(s == 0)
    def _out_tail():
        pltpu.sync_copy(acc.at[pl.ds(NS * ROWS_PER_TILE, ROWS_TAIL)],
                        out_hbm.at[pl.ds(c * N + NS * ROWS_PER_TILE,
                                         ROWS_TAIL)])


def _mlp_body(x_ref, p0_ref, p1_ref, w1_ref, b1_ref, w2_ref, b2_ref, o_ref):
    h = x_ref[...] + p0_ref[...] + p1_ref[...]
    t = jnp.dot(h, w1_ref[...], preferred_element_type=jnp.float32)
    t = jnp.maximum(t + b1_ref[...], 0.0)
    o_ref[...] = (jnp.dot(t, w2_ref[...], preferred_element_type=jnp.float32)
                  + b2_ref[...])


BM = 1000


def _tc_mlp(h, parts, W1, b1, W2, b2):
    nblk = N // BM
    return pl.pallas_call(
        _mlp_body,
        grid=(nblk,),
        in_specs=[
            pl.BlockSpec((BM, D), lambda i: (i, 0)),
            pl.BlockSpec((BM, D), lambda i: (i, 0)),
            pl.BlockSpec((BM, D), lambda i: (i + N // BM, 0)),
            pl.BlockSpec((D, D), lambda i: (0, 0)),
            pl.BlockSpec((1, D), lambda i: (0, 0)),
            pl.BlockSpec((D, D), lambda i: (0, 0)),
            pl.BlockSpec((1, D), lambda i: (0, 0)),
        ],
        out_specs=pl.BlockSpec((BM, D), lambda i: (i, 0)),
        out_shape=jax.ShapeDtypeStruct((N, D), jnp.float32),
    )(h, parts, parts, W1, b1, W2, b2)


def kernel(x, edge_index, batch, W1_0, b1_0, W2_0, b2_0,
           W1_1, b1_1, W2_1, b2_1, W1_2, b1_2, W2_2, b2_2):
    src = edge_index[0]
    dst = edge_index[1]
    zeros = jnp.zeros((N, D), jnp.float32)
    params = [(W1_0, b1_0, W2_0, b2_0),
              (W1_1, b1_1, W2_1, b2_1),
              (W1_2, b1_2, W2_2, b2_2)]
    h = x
    for (W1, b1, W2, b2) in params:
        parts = _sc_agg(h, src, dst, zeros)
        h = _tc_mlp(h, parts, W1, b1.reshape(1, D), W2, b2.reshape(1, D))
    return h


# speedup vs baseline: 11.4923x; 2.2961x over previous
import functools

import jax
import jax.numpy as jnp
from jax import lax
from jax.experimental import pallas as pl
from jax.experimental.pallas import tpu as pltpu
from jax.experimental.pallas import tpu_sc as plsc

N = 10000
E = 320000
D = 128

NC = 2
NS = 16
NW = NC * NS
EDGES_PER_W = E // NW
CHUNK = 128
NCHUNK = EDGES_PER_W // CHUNK
TAIL = EDGES_PER_W - NCHUNK * CHUNK
ROWS_PER_TILE = 624
ROWS_TAIL = N - NS * ROWS_PER_TILE

_mesh = plsc.VectorSubcoreMesh(core_axis_name="c", subcore_axis_name="s")


@functools.partial(
    pl.kernel,
    out_type=jax.ShapeDtypeStruct((NC * N, D), jnp.float32),
    mesh=_mesh,
    scratch_types=[
        pltpu.VMEM((CHUNK,), jnp.int32),
        pltpu.VMEM((CHUNK,), jnp.int32),
        pltpu.VMEM((CHUNK,), jnp.int32),
        pltpu.VMEM((CHUNK,), jnp.int32),
        pltpu.VMEM((TAIL,), jnp.int32),
        pltpu.VMEM((TAIL,), jnp.int32),
        pltpu.VMEM((CHUNK, D), jnp.float32),
        pltpu.VMEM((CHUNK, D), jnp.float32),
        pltpu.VMEM((TAIL, D), jnp.float32),
        pltpu.VMEM_SHARED((N, D), jnp.float32),
        pltpu.SemaphoreType.DMA,
        pltpu.SemaphoreType.DMA,
        pltpu.SemaphoreType.DMA,
        pltpu.SemaphoreType.DMA,
    ],
)
def _sc_agg(h_hbm, src_hbm, dst_hbm, zeros_hbm, out_hbm,
            sidx_a, didx_a, sidx_b, didx_b, sidx_t, didx_t,
            rows_a, rows_b, rows_t, acc,
            gsem_a, gsem_b, isem_a, isem_b):
    c = lax.axis_index("c")
    s = lax.axis_index("s")
    wid = c * NS + s
    ebase = wid * EDGES_PER_W

    def load_idx(i, sbuf, dbuf, sem):
        off = pl.multiple_of(ebase + i * CHUNK, 8)
        pltpu.async_copy(src_hbm.at[pl.ds(off, CHUNK)], sbuf, sem)
        pltpu.async_copy(dst_hbm.at[pl.ds(off, CHUNK)], dbuf, sem)

    def wait_idx(sbuf, dbuf, sem):
        pltpu.make_async_copy(src_hbm.at[pl.ds(0, CHUNK)], sbuf, sem).wait()
        pltpu.make_async_copy(dst_hbm.at[pl.ds(0, CHUNK)], dbuf, sem).wait()

    r0 = s * ROWS_PER_TILE
    pltpu.sync_copy(zeros_hbm.at[pl.ds(r0, ROWS_PER_TILE)],
                    acc.at[pl.ds(r0, ROWS_PER_TILE)])

    @pl.when(s == 0)
    def _zero_tail():
        pltpu.sync_copy(zeros_hbm.at[pl.ds(NS * ROWS_PER_TILE, ROWS_TAIL)],
                        acc.at[pl.ds(NS * ROWS_PER_TILE, ROWS_TAIL)])

    plsc.subcore_barrier()

    load_idx(0, sidx_a, didx_a, isem_a)
    wait_idx(sidx_a, didx_a, isem_a)
    pltpu.async_copy(h_hbm.at[sidx_a], rows_a, gsem_a)
    load_idx(1, sidx_b, didx_b, isem_b)

    def halfstep(i, sidx0, didx0, rows0, gsem0, isem0,
                 sidx1, didx1, rows1, gsem1, isem1):
        wait_idx(sidx1, didx1, isem1)
        pltpu.async_copy(h_hbm.at[sidx1], rows1, gsem1)
        pltpu.make_async_copy(h_hbm.at[sidx0], rows0, gsem0).wait()
        pltpu.sync_copy(rows0, acc.at[didx0], add=True)

        @pl.when(i + 2 < NCHUNK)
        def _refill():
            load_idx(i + 2, sidx0, didx0, isem0)

    def step(j, carry):
        halfstep(2 * j, sidx_a, didx_a, rows_a, gsem_a, isem_a,
                 sidx_b, didx_b, rows_b, gsem_b, isem_b)
        halfstep(2 * j + 1, sidx_b, didx_b, rows_b, gsem_b, isem_b,
                 sidx_a, didx_a, rows_a, gsem_a, isem_a)
        return carry

    lax.fori_loop(0, NCHUNK // 2 - 1, step, 0)

    i0 = NCHUNK - 2
    wait_idx(sidx_b, didx_b, isem_b)
    pltpu.async_copy(h_hbm.at[sidx_b], rows_b, gsem_b)
    pltpu.make_async_copy(h_hbm.at[sidx_a], rows_a, gsem_a).wait()
    pltpu.sync_copy(rows_a, acc.at[didx_a], add=True)
    toff = pl.multiple_of(ebase + NCHUNK * CHUNK, 8)
    pltpu.async_copy(src_hbm.at[pl.ds(toff, TAIL)], sidx_t, isem_a)
    pltpu.async_copy(dst_hbm.at[pl.ds(toff, TAIL)], didx_t, isem_a)
    pltpu.make_async_copy(src_hbm.at[pl.ds(0, TAIL)], sidx_t, isem_a).wait()
    pltpu.make_async_copy(dst_hbm.at[pl.ds(0, TAIL)], didx_t, isem_a).wait()
    pltpu.async_copy(h_hbm.at[sidx_t], rows_t, gsem_a)
    pltpu.make_async_copy(h_hbm.at[sidx_b], rows_b, gsem_b).wait()
    pltpu.sync_copy(rows_b, acc.at[didx_b], add=True)
    pltpu.make_async_copy(h_hbm.at[sidx_t], rows_t, gsem_a).wait()
    pltpu.sync_copy(rows_t, acc.at[didx_t], add=True)

    plsc.subcore_barrier()

    pltpu.sync_copy(acc.at[pl.ds(r0, ROWS_PER_TILE)],
                    out_hbm.at[pl.ds(c * N + r0, ROWS_PER_TILE)])

    @pl.when(s == 0)
    def _out_tail():
        pltpu.sync_copy(acc.at[pl.ds(NS * ROWS_PER_TILE, ROWS_TAIL)],
                        out_hbm.at[pl.ds(c * N + NS * ROWS_PER_TILE,
                                         ROWS_TAIL)])


def _mlp_body(x_ref, p0_ref, p1_ref, w1_ref, b1_ref, w2_ref, b2_ref, o_ref):
    h = x_ref[...] + p0_ref[...] + p1_ref[...]
    t = jnp.dot(h, w1_ref[...], preferred_element_type=jnp.float32)
    t = jnp.maximum(t + b1_ref[...], 0.0)
    o_ref[...] = (jnp.dot(t, w2_ref[...], preferred_element_type=jnp.float32)
                  + b2_ref[...])


BM = 1000


def _tc_mlp(h, parts, W1, b1, W2, b2):
    nblk = N // BM
    return pl.pallas_call(
        _mlp_body,
        grid=(nblk,),
        in_specs=[
            pl.BlockSpec((BM, D), lambda i: (i, 0)),
            pl.BlockSpec((BM, D), lambda i: (i, 0)),
            pl.BlockSpec((BM, D), lambda i: (i + N // BM, 0)),
            pl.BlockSpec((D, D), lambda i: (0, 0)),
            pl.BlockSpec((1, D), lambda i: (0, 0)),
            pl.BlockSpec((D, D), lambda i: (0, 0)),
            pl.BlockSpec((1, D), lambda i: (0, 0)),
        ],
        out_specs=pl.BlockSpec((BM, D), lambda i: (i, 0)),
        out_shape=jax.ShapeDtypeStruct((N, D), jnp.float32),
    )(h, parts, parts, W1, b1, W2, b2)


def kernel(x, edge_index, batch, W1_0, b1_0, W2_0, b2_0,
           W1_1, b1_1, W2_1, b2_1, W1_2, b1_2, W2_2, b2_2):
    src = edge_index[0]
    dst = edge_index[1]
    zeros = jnp.zeros((N, D), jnp.float32)
    params = [(W1_0, b1_0, W2_0, b2_0),
              (W1_1, b1_1, W2_1, b2_1),
              (W1_2, b1_2, W2_2, b2_2)]
    h = x
    for (W1, b1, W2, b2) in params:
        parts = _sc_agg(h, src, dst, zeros)
        h = _tc_mlp(h, parts, W1, b1.reshape(1, D), W2, b2.reshape(1, D))
    return h


# speedup vs baseline: 12.6820x; 1.1035x over previous
import functools

import jax
import jax.numpy as jnp
from jax import lax
from jax.experimental import pallas as pl
from jax.experimental.pallas import tpu as pltpu
from jax.experimental.pallas import tpu_sc as plsc

N = 10000
E = 320000
D = 128

NC = 2
NS = 16
NW = NC * NS
EDGES_PER_W = E // NW
CHUNK = 72
NCHUNK = 139
EDGES_PAD = NCHUNK * CHUNK
NACC = N + 8
RING = 4
ROWS_PER_TILE = 624
ROWS_TAIL = N - NS * ROWS_PER_TILE

_mesh = plsc.VectorSubcoreMesh(core_axis_name="c", subcore_axis_name="s")


@functools.partial(
    pl.kernel,
    out_type=jax.ShapeDtypeStruct((NC * N, D), jnp.float32),
    mesh=_mesh,
    scratch_types=[
        pltpu.VMEM((EDGES_PAD,), jnp.int32),
        [pltpu.VMEM((CHUNK,), jnp.int32) for _ in range(RING)],
        [pltpu.VMEM((CHUNK, D), jnp.float32) for _ in range(RING)],
        pltpu.VMEM_SHARED((NACC, D), jnp.float32),
        [pltpu.SemaphoreType.DMA for _ in range(RING)],
        [pltpu.SemaphoreType.DMA for _ in range(RING)],
        [pltpu.SemaphoreType.DMA for _ in range(RING)],
    ],
)
def _sc_agg(h_hbm, src_hbm, dst_hbm, zeros_hbm, out_hbm,
            sidx, didx, rows, acc, gsems, ssems, dsems):
    c = lax.axis_index("c")
    s = lax.axis_index("s")
    wid = c * NS + s
    ebase = wid * EDGES_PAD

    pltpu.async_copy(src_hbm.at[pl.ds(ebase, EDGES_PAD)], sidx, gsems[0])

    r0 = s * ROWS_PER_TILE
    pltpu.sync_copy(zeros_hbm.at[pl.ds(r0, ROWS_PER_TILE)],
                    acc.at[pl.ds(r0, ROWS_PER_TILE)])

    @pl.when(s == 0)
    def _zero_tail():
        pltpu.sync_copy(zeros_hbm.at[pl.ds(NS * ROWS_PER_TILE, ROWS_TAIL)],
                        acc.at[pl.ds(NS * ROWS_PER_TILE, ROWS_TAIL)])

    pltpu.make_async_copy(src_hbm.at[pl.ds(0, EDGES_PAD)], sidx,
                          gsems[0]).wait()
    plsc.subcore_barrier()

    def refill_didx(i, k):
        pltpu.async_copy(dst_hbm.at[pl.ds(ebase + i * CHUNK, CHUNK)],
                         didx[k], dsems[k])

    def wait_didx(k):
        pltpu.make_async_copy(dst_hbm.at[pl.ds(0, CHUNK)], didx[k],
                              dsems[k]).wait()

    def issue_gather(i, k):
        pltpu.async_copy(h_hbm.at[sidx.at[pl.ds(i * CHUNK, CHUNK)]],
                         rows[k], gsems[k])

    def wait_gather(k):
        pltpu.make_async_copy(h_hbm.at[sidx.at[pl.ds(0, CHUNK)]],
                              rows[k], gsems[k]).wait()

    def issue_scatter(k):
        pltpu.async_copy(rows[k], acc.at[didx[k]], ssems[k], add=True)

    def wait_scatter(k):
        pltpu.make_async_copy(rows[k], acc.at[didx[k]], ssems[k]).wait()

    refill_didx(0, 0)
    refill_didx(1, 1)
    issue_gather(0, 0)
    issue_gather(1, 1)

    def step(i, k):
        k2 = (k + 2) % RING

        @pl.when(i >= 2)
        def _drain():
            wait_scatter(k2)

        @pl.when(i + 2 < NCHUNK)
        def _prep():
            refill_didx(i + 2, k2)
            issue_gather(i + 2, k2)

        wait_gather(k)
        wait_didx(k)
        issue_scatter(k)

    step(0, 0)
    step(1, 1)

    def loop_body(j, carry):
        i = 2 + 4 * j
        step(i + 0, 2)
        step(i + 1, 3)
        step(i + 2, 0)
        step(i + 3, 1)
        return carry

    lax.fori_loop(0, (NCHUNK - 3) // 4, loop_body, 0)
    step(NCHUNK - 1, (NCHUNK - 1) % RING)
    wait_scatter((NCHUNK - 2) % RING)
    wait_scatter((NCHUNK - 1) % RING)

    plsc.subcore_barrier()

    pltpu.sync_copy(acc.at[pl.ds(r0, ROWS_PER_TILE)],
                    out_hbm.at[pl.ds(c * N + r0, ROWS_PER_TILE)])

    @pl.when(s == 0)
    def _out_tail():
        pltpu.sync_copy(acc.at[pl.ds(NS * ROWS_PER_TILE, ROWS_TAIL)],
                        out_hbm.at[pl.ds(c * N + NS * ROWS_PER_TILE,
                                         ROWS_TAIL)])


def _mlp_body(x_ref, p0_ref, p1_ref, w1_ref, b1_ref, w2_ref, b2_ref, o_ref):
    h = x_ref[...] + p0_ref[...] + p1_ref[...]
    t = jnp.dot(h, w1_ref[...], preferred_element_type=jnp.float32)
    t = jnp.maximum(t + b1_ref[...], 0.0)
    o_ref[...] = (jnp.dot(t, w2_ref[...], preferred_element_type=jnp.float32)
                  + b2_ref[...])


BM = 1000


def _tc_mlp(h, parts, W1, b1, W2, b2):
    nblk = N // BM
    return pl.pallas_call(
        _mlp_body,
        grid=(nblk,),
        in_specs=[
            pl.BlockSpec((BM, D), lambda i: (i, 0)),
            pl.BlockSpec((BM, D), lambda i: (i, 0)),
            pl.BlockSpec((BM, D), lambda i: (i + N // BM, 0)),
            pl.BlockSpec((D, D), lambda i: (0, 0)),
            pl.BlockSpec((1, D), lambda i: (0, 0)),
            pl.BlockSpec((D, D), lambda i: (0, 0)),
            pl.BlockSpec((1, D), lambda i: (0, 0)),
        ],
        out_specs=pl.BlockSpec((BM, D), lambda i: (i, 0)),
        out_shape=jax.ShapeDtypeStruct((N, D), jnp.float32),
    )(h, parts, parts, W1, b1, W2, b2)


def kernel(x, edge_index, batch, W1_0, b1_0, W2_0, b2_0,
           W1_1, b1_1, W2_1, b2_1, W1_2, b1_2, W2_2, b2_2):
    pad = EDGES_PAD - EDGES_PER_W
    src = jnp.pad(edge_index[0].reshape(NW, EDGES_PER_W),
                  ((0, 0), (0, pad))).reshape(-1)
    dst = jnp.pad(edge_index[1].reshape(NW, EDGES_PER_W),
                  ((0, 0), (0, pad)), constant_values=N).reshape(-1)
    zeros = jnp.zeros((N, D), jnp.float32)
    params = [(W1_0, b1_0, W2_0, b2_0),
              (W1_1, b1_1, W2_1, b2_1),
              (W1_2, b1_2, W2_2, b2_2)]
    h = x
    for (W1, b1, W2, b2) in params:
        parts = _sc_agg(h, src, dst, zeros)
        h = _tc_mlp(h, parts, W1, b1.reshape(1, D), W2, b2.reshape(1, D))
    return h


# speedup vs baseline: 14.5290x; 1.1456x over previous
import functools

import jax
import jax.numpy as jnp
from jax import lax
from jax.experimental import pallas as pl
from jax.experimental.pallas import tpu as pltpu
from jax.experimental.pallas import tpu_sc as plsc

N = 10000
E = 320000
D = 128

NC = 2
NS = 16
NW = NC * NS
EDGES_PER_W = E // NW
CHUNK = 80
NCHUNK = EDGES_PER_W // CHUNK
RING = 4
ROWS_PER_TILE = 624
ROWS_TAIL = N - NS * ROWS_PER_TILE

_mesh = plsc.VectorSubcoreMesh(core_axis_name="c", subcore_axis_name="s")


@functools.partial(
    pl.kernel,
    out_type=jax.ShapeDtypeStruct((NC * N, D), jnp.float32),
    mesh=_mesh,
    scratch_types=[
        [pltpu.VMEM((CHUNK,), jnp.int32) for _ in range(RING)],
        [pltpu.VMEM((CHUNK,), jnp.int32) for _ in range(RING)],
        [pltpu.VMEM((CHUNK, D), jnp.float32) for _ in range(RING)],
        pltpu.VMEM_SHARED((N, D), jnp.float32),
        [pltpu.SemaphoreType.DMA for _ in range(RING)],
        [pltpu.SemaphoreType.DMA for _ in range(RING)],
        [pltpu.SemaphoreType.DMA for _ in range(RING)],
        [pltpu.SemaphoreType.DMA for _ in range(RING)],
        pltpu.SemaphoreType.DMA,
    ],
)
def _sc_agg(h_hbm, src_hbm, dst_hbm, zeros_hbm, out_hbm,
            sidx, didx, rows, acc, gsems, ssems, isems, dsems, zsem):
    c = lax.axis_index("c")
    s = lax.axis_index("s")
    wid = c * NS + s
    ebase = wid * EDGES_PER_W

    def refill_sidx(i, k):
        pltpu.async_copy(src_hbm.at[pl.ds(ebase + i * CHUNK, CHUNK)],
                         sidx[k], isems[k])

    def wait_sidx(k):
        pltpu.make_async_copy(src_hbm.at[pl.ds(0, CHUNK)], sidx[k],
                              isems[k]).wait()

    def refill_didx(i, k):
        pltpu.async_copy(dst_hbm.at[pl.ds(ebase + i * CHUNK, CHUNK)],
                         didx[k], dsems[k])

    def wait_didx(k):
        pltpu.make_async_copy(dst_hbm.at[pl.ds(0, CHUNK)], didx[k],
                              dsems[k]).wait()

    def issue_gather(i, k):
        pltpu.async_copy(h_hbm.at[sidx[k]], rows[k], gsems[k])

    def wait_gather(k):
        pltpu.make_async_copy(h_hbm.at[sidx[k]], rows[k], gsems[k]).wait()

    def issue_scatter(k):
        pltpu.async_copy(rows[k], acc.at[didx[k]], ssems[k], add=True)

    def wait_scatter(k):
        pltpu.make_async_copy(rows[k], acc.at[didx[k]], ssems[k]).wait()

    r0 = s * ROWS_PER_TILE
    pltpu.async_copy(zeros_hbm.at[pl.ds(r0, ROWS_PER_TILE)],
                     acc.at[pl.ds(r0, ROWS_PER_TILE)], zsem)

    @pl.when(s == 0)
    def _zero_tail():
        pltpu.async_copy(zeros_hbm.at[pl.ds(NS * ROWS_PER_TILE, ROWS_TAIL)],
                         acc.at[pl.ds(NS * ROWS_PER_TILE, ROWS_TAIL)], zsem)

    refill_sidx(0, 0)
    refill_sidx(1, 1)
    refill_sidx(2, 2)
    refill_didx(0, 0)
    refill_didx(1, 1)
    wait_sidx(0)
    issue_gather(0, 0)
    wait_sidx(1)
    issue_gather(1, 1)

    pltpu.make_async_copy(zeros_hbm.at[pl.ds(0, ROWS_PER_TILE)],
                          acc.at[pl.ds(0, ROWS_PER_TILE)], zsem).wait()

    @pl.when(s == 0)
    def _zero_tail_wait():
        pltpu.make_async_copy(zeros_hbm.at[pl.ds(0, ROWS_TAIL)],
                              acc.at[pl.ds(0, ROWS_TAIL)], zsem).wait()

    plsc.subcore_barrier()

    def step(i, k):
        k2 = (k + 2) % RING
        k3 = (k + 3) % RING

        @pl.when(i >= 2)
        def _drain():
            wait_scatter(k2)

        @pl.when(i + 2 < NCHUNK)
        def _prep():
            refill_didx(i + 2, k2)
            wait_sidx(k2)
            issue_gather(i + 2, k2)

        @pl.when(i + 3 < NCHUNK)
        def _prefetch_sidx():
            refill_sidx(i + 3, k3)

        wait_gather(k)
        wait_didx(k)
        issue_scatter(k)

    step(0, 0)
    step(1, 1)

    def loop_body(j, carry):
        i = 2 + 4 * j
        step(i + 0, 2)
        step(i + 1, 3)
        step(i + 2, 0)
        step(i + 3, 1)
        return carry

    lax.fori_loop(0, (NCHUNK - 5) // 4, loop_body, 0)
    step(NCHUNK - 3, (NCHUNK - 3) % RING)
    step(NCHUNK - 2, (NCHUNK - 2) % RING)
    step(NCHUNK - 1, (NCHUNK - 1) % RING)
    wait_scatter((NCHUNK - 2) % RING)
    wait_scatter((NCHUNK - 1) % RING)

    plsc.subcore_barrier()

    pltpu.sync_copy(acc.at[pl.ds(r0, ROWS_PER_TILE)],
                    out_hbm.at[pl.ds(c * N + r0, ROWS_PER_TILE)])

    @pl.when(s == 0)
    def _out_tail():
        pltpu.sync_copy(acc.at[pl.ds(NS * ROWS_PER_TILE, ROWS_TAIL)],
                        out_hbm.at[pl.ds(c * N + NS * ROWS_PER_TILE,
                                         ROWS_TAIL)])


def _mlp_body(x_ref, p0_ref, p1_ref, w1_ref, b1_ref, w2_ref, b2_ref, o_ref):
    h = x_ref[...] + p0_ref[...] + p1_ref[...]
    t = jnp.dot(h, w1_ref[...], preferred_element_type=jnp.float32)
    t = jnp.maximum(t + b1_ref[...], 0.0)
    o_ref[...] = (jnp.dot(t, w2_ref[...], preferred_element_type=jnp.float32)
                  + b2_ref[...])


BM = 1000


def _tc_mlp(h, parts, W1, b1, W2, b2):
    nblk = N // BM
    return pl.pallas_call(
        _mlp_body,
        grid=(nblk,),
        in_specs=[
            pl.BlockSpec((BM, D), lambda i: (i, 0)),
            pl.BlockSpec((BM, D), lambda i: (i, 0)),
            pl.BlockSpec((BM, D), lambda i: (i + N // BM, 0)),
            pl.BlockSpec((D, D), lambda i: (0, 0)),
            pl.BlockSpec((1, D), lambda i: (0, 0)),
            pl.BlockSpec((D, D), lambda i: (0, 0)),
            pl.BlockSpec((1, D), lambda i: (0, 0)),
        ],
        out_specs=pl.BlockSpec((BM, D), lambda i: (i, 0)),
        out_shape=jax.ShapeDtypeStruct((N, D), jnp.float32),
    )(h, parts, parts, W1, b1, W2, b2)


def kernel(x, edge_index, batch, W1_0, b1_0, W2_0, b2_0,
           W1_1, b1_1, W2_1, b2_1, W1_2, b1_2, W2_2, b2_2):
    src = edge_index[0]
    dst = edge_index[1]
    zeros = jnp.zeros((N, D), jnp.float32)
    params = [(W1_0, b1_0, W2_0, b2_0),
              (W1_1, b1_1, W2_1, b2_1),
              (W1_2, b1_2, W2_2, b2_2)]
    h = x
    for (W1, b1, W2, b2) in params:
        parts = _sc_agg(h, src, dst, zeros)
        h = _tc_mlp(h, parts, W1, b1.reshape(1, D), W2, b2.reshape(1, D))
    return h
